# baseline (speedup 1.0000x reference)
import functools

import jax
import jax.numpy as jnp
from jax import lax
from jax.experimental import pallas as pl
from jax.experimental.pallas import tpu as pltpu
from jax.experimental.pallas import tpu_sc as plsc

N = 10000
E = 160000
EP = 163840
D = 512
NQ = 4
C = 256
GROUPS = 10
TILE_E = 10240
NH = 5120
DUMMY = NH
ACCR = 5248
ZR = ACCR // 16

_MESH = plsc.VectorSubcoreMesh(core_axis_name="c", subcore_axis_name="s")


def _mm_slabs(x, wt, bm):
    M, K = x.shape

    def body(x_ref, w_ref, o_ref):
        o_ref[0] = jnp.dot(x_ref[...], w_ref[...],
                           preferred_element_type=jnp.float32)

    return pl.pallas_call(
        body,
        grid=(M // bm, NQ),
        in_specs=[
            pl.BlockSpec((bm, K), lambda i, q: (i, 0)),
            pl.BlockSpec((K, 128), lambda i, q: (0, q)),
        ],
        out_specs=pl.BlockSpec((1, bm, 128), lambda i, q: (q, i, 0)),
        out_shape=jax.ShapeDtypeStruct((NQ, M, 128), jnp.float32),
    )(x, wt)


def _mm_w3(s4, wt, bm):
    def body(s_ref, w_ref, o_ref):
        s = s_ref[...]
        x = jnp.concatenate([s[0], s[1], s[2], s[3]], axis=1)
        o_ref[0] = jnp.dot(x, w_ref[...], preferred_element_type=jnp.float32)

    return pl.pallas_call(
        body,
        grid=(N // bm, NQ),
        in_specs=[
            pl.BlockSpec((NQ, bm, 128), lambda i, q: (0, i, 0)),
            pl.BlockSpec((D, 128), lambda i, q: (0, q)),
        ],
        out_specs=pl.BlockSpec((1, bm, 128), lambda i, q: (q, i, 0)),
        out_shape=jax.ShapeDtypeStruct((NQ, N, 128), jnp.float32),
    )(s4, wt)


def _mm_final(nf, s4, u1t, u2t, bm):
    def body(nf_ref, s_ref, u1_ref, u2_ref, o_ref):
        s = s_ref[...]
        x = jnp.concatenate([s[0], s[1], s[2], s[3]], axis=1)
        acc = jnp.dot(nf_ref[...], u1_ref[...],
                      preferred_element_type=jnp.float32)
        acc = acc + jnp.dot(x, u2_ref[...], preferred_element_type=jnp.float32)
        o_ref[...] = jnp.maximum(acc, 0.0)

    return pl.pallas_call(
        body,
        grid=(N // bm, NQ),
        in_specs=[
            pl.BlockSpec((bm, 256), lambda i, q: (i, 0)),
            pl.BlockSpec((NQ, bm, 128), lambda i, q: (0, i, 0)),
            pl.BlockSpec((256, 128), lambda i, q: (0, q)),
            pl.BlockSpec((D, 128), lambda i, q: (0, q)),
        ],
        out_specs=pl.BlockSpec((bm, 128), lambda i, q: (i, q)),
        out_shape=jax.ShapeDtypeStruct((N, D), jnp.float32),
    )(nf, s4, u1t, u2t)


def _stage_idx(src, soff, dst, off):
    for t in range(8):
        dst[pl.ds(t * 16, 16)] = src[pl.ds(soff + t * 16, 16)] + off


def _stage_lidx(src, soff, dst, lo):
    for t in range(8):
        rel = src[pl.ds(soff + t * 16, 16)] - lo
        ok = (rel >= 0) & (rel < NH)
        dst[pl.ds(t * 16, 16)] = jnp.where(ok, rel, DUMMY)


@functools.partial(
    pl.kernel,
    out_type=jax.ShapeDtypeStruct((NQ * EP, 128), jnp.float32),
    mesh=_MESH,
    scratch_types=[
        pltpu.VMEM((1024,), jnp.int32),
        pltpu.VMEM((128,), jnp.int32),
        pltpu.VMEM((128,), jnp.int32),
        pltpu.VMEM((C, 128), jnp.float32),
        pltpu.VMEM((C, 128), jnp.float32),
        pltpu.SemaphoreType.DMA,
        pltpu.SemaphoreType.DMA,
    ],
)
def _sc_base(p4, q4, from1d, base, fg, f0, f1, pbuf, qbuf, sem0, sem1):
    c = lax.axis_index("c")
    s = lax.axis_index("s")
    for j in range(2):
        q = 2 * c + j
        pbase = q * N
        ebase = q * EP

        def group(g, _):
            pltpu.sync_copy(from1d.at[pl.ds(s * TILE_E + g * 1024, 1024)], fg)
            for m in range(4):
                e0 = s * TILE_E + (g * 4 + m) * C
                _stage_idx(fg, m * C, f0, pbase)
                _stage_idx(fg, m * C + 128, f1, pbase)
                g0 = pltpu.async_copy(p4.at[f0], pbuf.at[pl.ds(0, 128)], sem0)
                g1 = pltpu.async_copy(p4.at[f1], pbuf.at[pl.ds(128, 128)],
                                      sem1)
                pltpu.sync_copy(q4.at[pl.ds(ebase + e0, C)], qbuf)
                g0.wait()
                g1.wait()

                def row(i, _):
                    for u in range(8):
                        sl = pl.ds(u * 16, 16)
                        qbuf[i, sl] = qbuf[i, sl] + pbuf[i, sl]
                    return 0

                lax.fori_loop(0, C, row, 0)
                pltpu.sync_copy(qbuf, base.at[pl.ds(ebase + e0, C)])
            return 0

        lax.fori_loop(0, GROUPS, group, 0)


@functools.partial(
    pl.kernel,
    out_type=jax.ShapeDtypeStruct((NQ * N, 128), jnp.float32),
    mesh=_MESH,
    scratch_types=[
        pltpu.VMEM((1024,), jnp.int32),
        pltpu.VMEM((1024,), jnp.int32),
        pltpu.VMEM((128,), jnp.int32),
        pltpu.VMEM((128,), jnp.int32),
        pltpu.VMEM((128,), jnp.int32),
        pltpu.VMEM((128,), jnp.int32),
        pltpu.VMEM((C, 128), jnp.float32),
        pltpu.VMEM((C, 128), jnp.float32),
        pltpu.VMEM_SHARED((ACCR, 128), jnp.float32),
        pltpu.SemaphoreType.DMA,
        pltpu.SemaphoreType.DMA,
    ],
)
def _sc_step(base, m4, from1d, to1d, zrows, out,
             fg, tg, f0, f1, t0, t1, bbuf, mbuf, acc, sem0, sem1):
    c = lax.axis_index("c")
    s = lax.axis_index("s")
    for j in range(2):
        q = 2 * c + j
        pbase = q * N
        ebase = q * EP
        for p in range(2):
            lo = p * NH
            pltpu.sync_copy(zrows, acc.at[pl.ds(s * ZR, ZR)])
            plsc.subcore_barrier()

            def group(g, _):
                pltpu.sync_copy(
                    from1d.at[pl.ds(s * TILE_E + g * 1024, 1024)], fg)
                pltpu.sync_copy(
                    to1d.at[pl.ds(s * TILE_E + g * 1024, 1024)], tg)
                for m in range(4):
                    e0 = s * TILE_E + (g * 4 + m) * C
                    _stage_idx(fg, m * C, f0, pbase)
                    _stage_idx(fg, m * C + 128, f1, pbase)
                    _stage_lidx(tg, m * C, t0, lo)
                    _stage_lidx(tg, m * C + 128, t1, lo)
                    g0 = pltpu.async_copy(m4.at[f0], mbuf.at[pl.ds(0, 128)],
                                          sem0)
                    g1 = pltpu.async_copy(m4.at[f1],
                                          mbuf.at[pl.ds(128, 128)], sem1)
                    pltpu.sync_copy(base.at[pl.ds(ebase + e0, C)], bbuf)
                    g0.wait()
                    g1.wait()

                    def row(i, _):
                        for u in range(8):
                            sl = pl.ds(u * 16, 16)
                            bbuf[i, sl] = jnp.maximum(
                                bbuf[i, sl] + mbuf[i, sl], 0.0)
                        return 0

                    lax.fori_loop(0, C, row, 0)
                    pltpu.sync_copy(bbuf.at[pl.ds(0, 128)], acc.at[t0],
                                    add=True)
                    pltpu.sync_copy(bbuf.at[pl.ds(128, 128)], acc.at[t1],
                                    add=True)
                return 0

            lax.fori_loop(0, GROUPS, group, 0)
            plsc.subcore_barrier()
            wr = 320 if p == 0 else 304
            pltpu.sync_copy(acc.at[pl.ds(s * wr, wr)],
                            out.at[pl.ds(pbase + p * NH + s * wr, wr)])
            if p == 1:
                @pl.when(s == 15)
                def _tail():
                    pltpu.sync_copy(
                        acc.at[pl.ds(16 * 304, 16)],
                        out.at[pl.ds(pbase + NH + 16 * 304, 16)])
            plsc.subcore_barrier()


@functools.partial(
    pl.kernel,
    out_type=jax.ShapeDtypeStruct((NQ * N, 128), jnp.float32),
    mesh=_MESH,
    scratch_types=[
        pltpu.VMEM((1024,), jnp.int32),
        pltpu.VMEM((128,), jnp.int32),
        pltpu.VMEM((128,), jnp.int32),
        pltpu.VMEM((C, 128), jnp.float32),
        pltpu.VMEM_SHARED((ACCR, 128), jnp.float32),
    ],
)
def _sc_step1(base, to1d, zrows, out, tg, t0, t1, bbuf, acc):
    c = lax.axis_index("c")
    s = lax.axis_index("s")
    for j in range(2):
        q = 2 * c + j
        pbase = q * N
        ebase = q * EP
        for p in range(2):
            lo = p * NH
            pltpu.sync_copy(zrows, acc.at[pl.ds(s * ZR, ZR)])
            plsc.subcore_barrier()

            def group(g, _):
                pltpu.sync_copy(
                    to1d.at[pl.ds(s * TILE_E + g * 1024, 1024)], tg)
                for m in range(4):
                    e0 = s * TILE_E + (g * 4 + m) * C
                    _stage_lidx(tg, m * C, t0, lo)
                    _stage_lidx(tg, m * C + 128, t1, lo)
                    pltpu.sync_copy(base.at[pl.ds(ebase + e0, C)], bbuf)

                    def row(i, _):
                        for u in range(8):
                            sl = pl.ds(u * 16, 16)
                            bbuf[i, sl] = jnp.maximum(bbuf[i, sl], 0.0)
                        return 0

                    lax.fori_loop(0, C, row, 0)
                    pltpu.sync_copy(bbuf.at[pl.ds(0, 128)], acc.at[t0],
                                    add=True)
                    pltpu.sync_copy(bbuf.at[pl.ds(128, 128)], acc.at[t1],
                                    add=True)
                return 0

            lax.fori_loop(0, GROUPS, group, 0)
            plsc.subcore_barrier()
            wr = 320 if p == 0 else 304
            pltpu.sync_copy(acc.at[pl.ds(s * wr, wr)],
                            out.at[pl.ds(pbase + p * NH + s * wr, wr)])
            if p == 1:
                @pl.when(s == 15)
                def _tail():
                    pltpu.sync_copy(
                        acc.at[pl.ds(16 * 304, 16)],
                        out.at[pl.ds(pbase + NH + 16 * 304, 16)])
            plsc.subcore_barrier()


def kernel(node_features, edge_features, edge_hiddens, edge_index,
           W1, W2, W3, U1, U2):
    del edge_hiddens
    from_n = edge_index[0]
    to_n = edge_index[1]
    from1d = jnp.pad(from_n, (0, EP - E))
    to1d = jnp.pad(to_n, (0, EP - E), constant_values=N)
    ef_p = jnp.pad(edge_features, ((0, EP - E), (0, 0)))
    zrows = jnp.zeros((ZR, 128), jnp.float32)

    p4 = _mm_slabs(node_features, W1.T, bm=400)
    q4 = _mm_slabs(ef_p, W2.T, bm=1024)
    base = _sc_base(p4.reshape(NQ * N, 128),
                    q4.reshape(NQ * EP, 128), from1d)

    s4 = _sc_step1(base, to1d, zrows)
    for _ in range(7):
        m4 = _mm_w3(s4.reshape(NQ, N, 128), W3.T, bm=400)
        s4 = _sc_step(base, m4.reshape(NQ * N, 128), from1d, to1d, zrows)

    return _mm_final(node_features, s4.reshape(NQ, N, 128),
                     U1.T, U2.T, bm=400)


# speedup vs baseline: 1.1950x; 1.1950x over previous
import functools

import jax
import jax.numpy as jnp
from jax import lax
from jax.experimental import pallas as pl
from jax.experimental.pallas import tpu as pltpu
from jax.experimental.pallas import tpu_sc as plsc

N = 10000
E = 160000
EP = 163840
D = 512
NQ = 4
C = 256
CC = 128
GROUPS = 10
TILE_E = 10240
NH = 5120
DUMMY = NH
ACCR = 5248
ZR = ACCR // 16

_MESH = plsc.VectorSubcoreMesh(core_axis_name="c", subcore_axis_name="s")


def _mm_slabs(x, wt, bm):
    M, K = x.shape

    def body(x_ref, w_ref, o_ref):
        o_ref[0] = jnp.dot(x_ref[...], w_ref[...],
                           preferred_element_type=jnp.float32)

    return pl.pallas_call(
        body,
        grid=(M // bm, NQ),
        in_specs=[
            pl.BlockSpec((bm, K), lambda i, q: (i, 0)),
            pl.BlockSpec((K, 128), lambda i, q: (0, q)),
        ],
        out_specs=pl.BlockSpec((1, bm, 128), lambda i, q: (q, i, 0)),
        out_shape=jax.ShapeDtypeStruct((NQ, M, 128), jnp.float32),
    )(x, wt)


def _mm_w3(s4, wt, bm):
    def body(s_ref, w_ref, o_ref):
        s = s_ref[...]
        x = jnp.concatenate([s[0], s[1], s[2], s[3]], axis=1)
        o_ref[0] = jnp.dot(x, w_ref[...], preferred_element_type=jnp.float32)

    return pl.pallas_call(
        body,
        grid=(N // bm, NQ),
        in_specs=[
            pl.BlockSpec((NQ, bm, 128), lambda i, q: (0, i, 0)),
            pl.BlockSpec((D, 128), lambda i, q: (0, q)),
        ],
        out_specs=pl.BlockSpec((1, bm, 128), lambda i, q: (q, i, 0)),
        out_shape=jax.ShapeDtypeStruct((NQ, N, 128), jnp.float32),
    )(s4, wt)


def _mm_final(nf, s4, u1t, u2t, bm):
    def body(nf_ref, s_ref, u1_ref, u2_ref, o_ref):
        s = s_ref[...]
        x = jnp.concatenate([s[0], s[1], s[2], s[3]], axis=1)
        acc = jnp.dot(nf_ref[...], u1_ref[...],
                      preferred_element_type=jnp.float32)
        acc = acc + jnp.dot(x, u2_ref[...], preferred_element_type=jnp.float32)
        o_ref[...] = jnp.maximum(acc, 0.0)

    return pl.pallas_call(
        body,
        grid=(N // bm, NQ),
        in_specs=[
            pl.BlockSpec((bm, 256), lambda i, q: (i, 0)),
            pl.BlockSpec((NQ, bm, 128), lambda i, q: (0, i, 0)),
            pl.BlockSpec((256, 128), lambda i, q: (0, q)),
            pl.BlockSpec((D, 128), lambda i, q: (0, q)),
        ],
        out_specs=pl.BlockSpec((bm, 128), lambda i, q: (i, q)),
        out_shape=jax.ShapeDtypeStruct((N, D), jnp.float32),
    )(nf, s4, u1t, u2t)


def _stage_idx(src, soff, dst, off):
    for t in range(8):
        dst[pl.ds(t * 16, 16)] = src[pl.ds(soff + t * 16, 16)] + off


def _stage_lidx(src, soff, dst, lo):
    for t in range(8):
        rel = src[pl.ds(soff + t * 16, 16)] - lo
        ok = (rel >= 0) & (rel < NH)
        dst[pl.ds(t * 16, 16)] = jnp.where(ok, rel, DUMMY)


@functools.partial(
    pl.kernel,
    out_type=jax.ShapeDtypeStruct((NQ * EP, 128), jnp.float32),
    mesh=_MESH,
    scratch_types=[
        pltpu.VMEM((1024,), jnp.int32),
        pltpu.VMEM((128,), jnp.int32),
        pltpu.VMEM((128,), jnp.int32),
        pltpu.VMEM((C, 128), jnp.float32),
        pltpu.VMEM((C, 128), jnp.float32),
        pltpu.SemaphoreType.DMA,
        pltpu.SemaphoreType.DMA,
    ],
)
def _sc_base(p4, q4, from1d, base, fg, f0, f1, pbuf, qbuf, sem0, sem1):
    c = lax.axis_index("c")
    s = lax.axis_index("s")
    for j in range(2):
        q = 2 * c + j
        pbase = q * N
        ebase = q * EP

        def group(g, _):
            pltpu.sync_copy(from1d.at[pl.ds(s * TILE_E + g * 1024, 1024)], fg)
            for m in range(4):
                e0 = s * TILE_E + (g * 4 + m) * C
                _stage_idx(fg, m * C, f0, pbase)
                _stage_idx(fg, m * C + 128, f1, pbase)
                g0 = pltpu.async_copy(p4.at[f0], pbuf.at[pl.ds(0, 128)], sem0)
                g1 = pltpu.async_copy(p4.at[f1], pbuf.at[pl.ds(128, 128)],
                                      sem1)
                pltpu.sync_copy(q4.at[pl.ds(ebase + e0, C)], qbuf)
                g0.wait()
                g1.wait()

                def row(i, _):
                    for u in range(8):
                        sl = pl.ds(u * 16, 16)
                        qbuf[i, sl] = qbuf[i, sl] + pbuf[i, sl]
                    return 0

                lax.fori_loop(0, C, row, 0)
                pltpu.sync_copy(qbuf, base.at[pl.ds(ebase + e0, C)])
            return 0

        lax.fori_loop(0, GROUPS, group, 0)


@functools.partial(
    pl.kernel,
    out_type=jax.ShapeDtypeStruct((NQ * N, 128), jnp.float32),
    mesh=_MESH,
    scratch_types=[
        pltpu.VMEM((1024,), jnp.int32),
        pltpu.VMEM((1024,), jnp.int32),
        pltpu.VMEM((128,), jnp.int32),
        pltpu.VMEM((128,), jnp.int32),
        pltpu.VMEM((128,), jnp.int32),
        pltpu.VMEM((128,), jnp.int32),
        pltpu.VMEM((CC, 128), jnp.float32),
        pltpu.VMEM((CC, 128), jnp.float32),
        pltpu.VMEM((CC, 128), jnp.float32),
        pltpu.VMEM((CC, 128), jnp.float32),
        pltpu.VMEM_SHARED((ACCR, 128), jnp.float32),
        pltpu.SemaphoreType.DMA,
        pltpu.SemaphoreType.DMA,
        pltpu.SemaphoreType.DMA,
        pltpu.SemaphoreType.DMA,
        pltpu.SemaphoreType.DMA,
        pltpu.SemaphoreType.DMA,
    ],
)
def _sc_step(base, m4, from1d, to1d, zrows, out,
             fg, tg, fA, fB, tA, tB, bbA, bbB, mbA, mbB, acc,
             gsA, gsB, bsA, bsB, ssA, ssB):
    c = lax.axis_index("c")
    s = lax.axis_index("s")
    sets = [(fA, tA, bbA, mbA, gsA, bsA, ssA),
            (fB, tB, bbB, mbB, gsB, bsB, ssB)]
    for j in range(2):
        q = 2 * c + j
        pbase = q * N
        ebase = q * EP
        for p in range(2):
            lo = p * NH
            pltpu.sync_copy(zrows, acc.at[pl.ds(s * ZR, ZR)])
            plsc.subcore_barrier()

            def group(g, _, pbase=pbase, ebase=ebase, lo=lo):
                eg = s * TILE_E + g * 1024
                pltpu.sync_copy(from1d.at[pl.ds(eg, 1024)], fg)
                pltpu.sync_copy(to1d.at[pl.ds(eg, 1024)], tg)
                f0, t0, bb0, mb0, gs0, bs0, _ss0 = sets[0]
                _stage_idx(fg, 0, f0, pbase)
                _stage_lidx(tg, 0, t0, lo)
                gd = [None, None]
                bd = [None, None]
                sd = [None, None]
                gd[0] = pltpu.async_copy(m4.at[f0], mb0, gs0)
                bd[0] = pltpu.async_copy(base.at[pl.ds(ebase + eg, CC)],
                                         bb0, bs0)
                for m in range(8):
                    S = m % 2
                    T = 1 - S
                    fS, tS, bbS, mbS, _g, _b, ssS = sets[S]
                    fT, tT, bbT, mbT, gsT, bsT, _s2 = sets[T]
                    if m < 7:
                        if sd[T] is not None:
                            sd[T].wait()
                        _stage_idx(fg, (m + 1) * CC, fT, pbase)
                        _stage_lidx(tg, (m + 1) * CC, tT, lo)
                        gd[T] = pltpu.async_copy(m4.at[fT], mbT, gsT)
                        bd[T] = pltpu.async_copy(
                            base.at[pl.ds(ebase + eg + (m + 1) * CC, CC)],
                            bbT, bsT)
                    gd[S].wait()
                    bd[S].wait()

                    def row(i, _, bbS=bbS, mbS=mbS):
                        for u in range(8):
                            sl = pl.ds(u * 16, 16)
                            bbS[i, sl] = jnp.maximum(
                                bbS[i, sl] + mbS[i, sl], 0.0)
                        return 0

                    lax.fori_loop(0, CC, row, 0)
                    sd[S] = pltpu.async_copy(bbS, acc.at[tS], ssS, add=True)
                sd[0].wait()
                sd[1].wait()
                return 0

            lax.fori_loop(0, GROUPS, group, 0)
            plsc.subcore_barrier()
            wr = 320 if p == 0 else 304
            pltpu.sync_copy(acc.at[pl.ds(s * wr, wr)],
                            out.at[pl.ds(pbase + p * NH + s * wr, wr)])
            if p == 1:
                @pl.when(s == 15)
                def _tail():
                    pltpu.sync_copy(
                        acc.at[pl.ds(16 * 304, 16)],
                        out.at[pl.ds(pbase + NH + 16 * 304, 16)])
            plsc.subcore_barrier()


@functools.partial(
    pl.kernel,
    out_type=jax.ShapeDtypeStruct((NQ * N, 128), jnp.float32),
    mesh=_MESH,
    scratch_types=[
        pltpu.VMEM((1024,), jnp.int32),
        pltpu.VMEM((128,), jnp.int32),
        pltpu.VMEM((128,), jnp.int32),
        pltpu.VMEM((CC, 128), jnp.float32),
        pltpu.VMEM((CC, 128), jnp.float32),
        pltpu.VMEM_SHARED((ACCR, 128), jnp.float32),
        pltpu.SemaphoreType.DMA,
        pltpu.SemaphoreType.DMA,
        pltpu.SemaphoreType.DMA,
        pltpu.SemaphoreType.DMA,
    ],
)
def _sc_step1(base, to1d, zrows, out,
              tg, tA, tB, bbA, bbB, acc, bsA, bsB, ssA, ssB):
    c = lax.axis_index("c")
    s = lax.axis_index("s")
    sets = [(tA, bbA, bsA, ssA), (tB, bbB, bsB, ssB)]
    for j in range(2):
        q = 2 * c + j
        pbase = q * N
        ebase = q * EP
        for p in range(2):
            lo = p * NH
            pltpu.sync_copy(zrows, acc.at[pl.ds(s * ZR, ZR)])
            plsc.subcore_barrier()

            def group(g, _, pbase=pbase, ebase=ebase, lo=lo):
                eg = s * TILE_E + g * 1024
                pltpu.sync_copy(to1d.at[pl.ds(eg, 1024)], tg)
                t0, bb0, bs0, _s0 = sets[0]
                _stage_lidx(tg, 0, t0, lo)
                bd = [None, None]
                sd = [None, None]
                bd[0] = pltpu.async_copy(base.at[pl.ds(ebase + eg, CC)],
                                         bb0, bs0)
                for m in range(8):
                    S = m % 2
                    T = 1 - S
                    tS, bbS, _b, ssS = sets[S]
                    tT, bbT, bsT, _s2 = sets[T]
                    if m < 7:
                        if sd[T] is not None:
                            sd[T].wait()
                        _stage_lidx(tg, (m + 1) * CC, tT, lo)
                        bd[T] = pltpu.async_copy(
                            base.at[pl.ds(ebase + eg + (m + 1) * CC, CC)],
                            bbT, bsT)
                    bd[S].wait()

                    def row(i, _, bbS=bbS):
                        for u in range(8):
                            sl = pl.ds(u * 16, 16)
                            bbS[i, sl] = jnp.maximum(bbS[i, sl], 0.0)
                        return 0

                    lax.fori_loop(0, CC, row, 0)
                    sd[S] = pltpu.async_copy(bbS, acc.at[tS], ssS, add=True)
                sd[0].wait()
                sd[1].wait()
                return 0

            lax.fori_loop(0, GROUPS, group, 0)
            plsc.subcore_barrier()
            wr = 320 if p == 0 else 304
            pltpu.sync_copy(acc.at[pl.ds(s * wr, wr)],
                            out.at[pl.ds(pbase + p * NH + s * wr, wr)])
            if p == 1:
                @pl.when(s == 15)
                def _tail():
                    pltpu.sync_copy(
                        acc.at[pl.ds(16 * 304, 16)],
                        out.at[pl.ds(pbase + NH + 16 * 304, 16)])
            plsc.subcore_barrier()


def kernel(node_features, edge_features, edge_hiddens, edge_index,
           W1, W2, W3, U1, U2):
    del edge_hiddens
    from_n = edge_index[0]
    to_n = edge_index[1]
    from1d = jnp.pad(from_n, (0, EP - E))
    to1d = jnp.pad(to_n, (0, EP - E), constant_values=N)
    ef_p = jnp.pad(edge_features, ((0, EP - E), (0, 0)))
    zrows = jnp.zeros((ZR, 128), jnp.float32)

    p4 = _mm_slabs(node_features, W1.T, bm=400)
    q4 = _mm_slabs(ef_p, W2.T, bm=1024)
    base = _sc_base(p4.reshape(NQ * N, 128),
                    q4.reshape(NQ * EP, 128), from1d)

    s4 = _sc_step1(base, to1d, zrows)
    for _ in range(7):
        m4 = _mm_w3(s4.reshape(NQ, N, 128), W3.T, bm=400)
        s4 = _sc_step(base, m4.reshape(NQ * N, 128), from1d, to1d, zrows)

    return _mm_final(node_features, s4.reshape(NQ, N, 128),
                     U1.T, U2.T, bm=400)


# speedup vs baseline: 1.2023x; 1.0061x over previous
import functools

import jax
import jax.numpy as jnp
from jax import lax
from jax.experimental import pallas as pl
from jax.experimental.pallas import tpu as pltpu
from jax.experimental.pallas import tpu_sc as plsc

N = 10000
E = 160000
EP = 163840
D = 512
NQ = 4
C = 256
CC = 128
GROUPS = 10
TILE_E = 10240
NH = 5120
DUMMY = NH
ACCR = 5248
ZR = ACCR // 16

_MESH = plsc.VectorSubcoreMesh(core_axis_name="c", subcore_axis_name="s")


def _mm_slabs(x, wt, bm):
    M, K = x.shape

    def body(x_ref, w_ref, o_ref):
        o_ref[0] = jnp.dot(x_ref[...], w_ref[...],
                           preferred_element_type=jnp.float32)

    return pl.pallas_call(
        body,
        grid=(M // bm, NQ),
        in_specs=[
            pl.BlockSpec((bm, K), lambda i, q: (i, 0)),
            pl.BlockSpec((K, 128), lambda i, q: (0, q)),
        ],
        out_specs=pl.BlockSpec((1, bm, 128), lambda i, q: (q, i, 0)),
        out_shape=jax.ShapeDtypeStruct((NQ, M, 128), jnp.float32),
    )(x, wt)


def _mm_w3(s4, wt, bm):
    def body(s_ref, w_ref, o_ref):
        s = s_ref[...]
        x = jnp.concatenate([s[0], s[1], s[2], s[3]], axis=1)
        o_ref[0] = jnp.dot(x, w_ref[...], preferred_element_type=jnp.float32)

    return pl.pallas_call(
        body,
        grid=(N // bm, NQ),
        in_specs=[
            pl.BlockSpec((NQ, bm, 128), lambda i, q: (0, i, 0)),
            pl.BlockSpec((D, 128), lambda i, q: (0, q)),
        ],
        out_specs=pl.BlockSpec((1, bm, 128), lambda i, q: (q, i, 0)),
        out_shape=jax.ShapeDtypeStruct((NQ, N, 128), jnp.float32),
    )(s4, wt)


def _mm_final(nf, s4, u1t, u2t, bm):
    def body(nf_ref, s_ref, u1_ref, u2_ref, o_ref):
        s = s_ref[...]
        x = jnp.concatenate([s[0], s[1], s[2], s[3]], axis=1)
        acc = jnp.dot(nf_ref[...], u1_ref[...],
                      preferred_element_type=jnp.float32)
        acc = acc + jnp.dot(x, u2_ref[...], preferred_element_type=jnp.float32)
        o_ref[...] = jnp.maximum(acc, 0.0)

    return pl.pallas_call(
        body,
        grid=(N // bm, NQ),
        in_specs=[
            pl.BlockSpec((bm, 256), lambda i, q: (i, 0)),
            pl.BlockSpec((NQ, bm, 128), lambda i, q: (0, i, 0)),
            pl.BlockSpec((256, 128), lambda i, q: (0, q)),
            pl.BlockSpec((D, 128), lambda i, q: (0, q)),
        ],
        out_specs=pl.BlockSpec((bm, 128), lambda i, q: (i, q)),
        out_shape=jax.ShapeDtypeStruct((N, D), jnp.float32),
    )(nf, s4, u1t, u2t)


def _stage_idx(src, soff, dst, off):
    for t in range(8):
        dst[pl.ds(t * 16, 16)] = src[pl.ds(soff + t * 16, 16)] + off


def _stage_lidx(src, soff, dst, lo):
    for t in range(8):
        rel = src[pl.ds(soff + t * 16, 16)] - lo
        ok = (rel >= 0) & (rel < NH)
        dst[pl.ds(t * 16, 16)] = jnp.where(ok, rel, DUMMY)


@functools.partial(
    pl.kernel,
    out_type=jax.ShapeDtypeStruct((NQ * EP, 128), jnp.float32),
    mesh=_MESH,
    scratch_types=[
        pltpu.VMEM((1024,), jnp.int32),
        pltpu.VMEM((128,), jnp.int32),
        pltpu.VMEM((128,), jnp.int32),
        pltpu.VMEM((C, 128), jnp.float32),
        pltpu.VMEM((C, 128), jnp.float32),
        pltpu.SemaphoreType.DMA,
        pltpu.SemaphoreType.DMA,
    ],
)
def _sc_base(p4, q4, from1d, base, fg, f0, f1, pbuf, qbuf, sem0, sem1):
    c = lax.axis_index("c")
    s = lax.axis_index("s")
    for j in range(2):
        q = 2 * c + j
        pbase = q * N
        ebase = q * EP

        def group(g, _):
            pltpu.sync_copy(from1d.at[pl.ds(s * TILE_E + g * 1024, 1024)], fg)
            for m in range(4):
                e0 = s * TILE_E + (g * 4 + m) * C
                _stage_idx(fg, m * C, f0, pbase)
                _stage_idx(fg, m * C + 128, f1, pbase)
                g0 = pltpu.async_copy(p4.at[f0], pbuf.at[pl.ds(0, 128)], sem0)
                g1 = pltpu.async_copy(p4.at[f1], pbuf.at[pl.ds(128, 128)],
                                      sem1)
                pltpu.sync_copy(q4.at[pl.ds(ebase + e0, C)], qbuf)
                g0.wait()
                g1.wait()

                @plsc.parallel_loop(0, C, step=1, unroll=4)
                def row(i):
                    for u in range(8):
                        sl = pl.ds(u * 16, 16)
                        qbuf[i, sl] = qbuf[i, sl] + pbuf[i, sl]
                pltpu.sync_copy(qbuf, base.at[pl.ds(ebase + e0, C)])
            return 0

        lax.fori_loop(0, GROUPS, group, 0)


@functools.partial(
    pl.kernel,
    out_type=jax.ShapeDtypeStruct((NQ * N, 128), jnp.float32),
    mesh=_MESH,
    scratch_types=[
        pltpu.VMEM((1024,), jnp.int32),
        pltpu.VMEM((1024,), jnp.int32),
        pltpu.VMEM((128,), jnp.int32),
        pltpu.VMEM((128,), jnp.int32),
        pltpu.VMEM((128,), jnp.int32),
        pltpu.VMEM((128,), jnp.int32),
        pltpu.VMEM((CC, 128), jnp.float32),
        pltpu.VMEM((CC, 128), jnp.float32),
        pltpu.VMEM((CC, 128), jnp.float32),
        pltpu.VMEM((CC, 128), jnp.float32),
        pltpu.VMEM_SHARED((ACCR, 128), jnp.float32),
        pltpu.SemaphoreType.DMA,
        pltpu.SemaphoreType.DMA,
        pltpu.SemaphoreType.DMA,
        pltpu.SemaphoreType.DMA,
        pltpu.SemaphoreType.DMA,
        pltpu.SemaphoreType.DMA,
    ],
)
def _sc_step(base, m4, from1d, to1d, zrows, out,
             fg, tg, fA, fB, tA, tB, bbA, bbB, mbA, mbB, acc,
             gsA, gsB, bsA, bsB, ssA, ssB):
    c = lax.axis_index("c")
    s = lax.axis_index("s")
    sets = [(fA, tA, bbA, mbA, gsA, bsA, ssA),
            (fB, tB, bbB, mbB, gsB, bsB, ssB)]
    for j in range(2):
        q = 2 * c + j
        pbase = q * N
        ebase = q * EP
        for p in range(2):
            lo = p * NH
            pltpu.sync_copy(zrows, acc.at[pl.ds(s * ZR, ZR)])
            plsc.subcore_barrier()

            def group(g, _, pbase=pbase, ebase=ebase, lo=lo):
                eg = s * TILE_E + g * 1024
                pltpu.sync_copy(from1d.at[pl.ds(eg, 1024)], fg)
                pltpu.sync_copy(to1d.at[pl.ds(eg, 1024)], tg)
                f0, t0, bb0, mb0, gs0, bs0, _ss0 = sets[0]
                _stage_idx(fg, 0, f0, pbase)
                _stage_lidx(tg, 0, t0, lo)
                gd = [None, None]
                bd = [None, None]
                sd = [None, None]
                gd[0] = pltpu.async_copy(m4.at[f0], mb0, gs0)
                bd[0] = pltpu.async_copy(base.at[pl.ds(ebase + eg, CC)],
                                         bb0, bs0)
                for m in range(8):
                    S = m % 2
                    T = 1 - S
                    fS, tS, bbS, mbS, _g, _b, ssS = sets[S]
                    fT, tT, bbT, mbT, gsT, bsT, _s2 = sets[T]
                    if m < 7:
                        if sd[T] is not None:
                            sd[T].wait()
                        _stage_idx(fg, (m + 1) * CC, fT, pbase)
                        _stage_lidx(tg, (m + 1) * CC, tT, lo)
                        gd[T] = pltpu.async_copy(m4.at[fT], mbT, gsT)
                        bd[T] = pltpu.async_copy(
                            base.at[pl.ds(ebase + eg + (m + 1) * CC, CC)],
                            bbT, bsT)
                    gd[S].wait()
                    bd[S].wait()

                    @plsc.parallel_loop(0, CC, step=1, unroll=4)
                    def row(i, bbS=bbS, mbS=mbS):
                        for u in range(8):
                            sl = pl.ds(u * 16, 16)
                            bbS[i, sl] = jnp.maximum(
                                bbS[i, sl] + mbS[i, sl], 0.0)
                    sd[S] = pltpu.async_copy(bbS, acc.at[tS], ssS, add=True)
                sd[0].wait()
                sd[1].wait()
                return 0

            lax.fori_loop(0, GROUPS, group, 0)
            plsc.subcore_barrier()
            wr = 320 if p == 0 else 304
            pltpu.sync_copy(acc.at[pl.ds(s * wr, wr)],
                            out.at[pl.ds(pbase + p * NH + s * wr, wr)])
            if p == 1:
                @pl.when(s == 15)
                def _tail():
                    pltpu.sync_copy(
                        acc.at[pl.ds(16 * 304, 16)],
                        out.at[pl.ds(pbase + NH + 16 * 304, 16)])
            plsc.subcore_barrier()


@functools.partial(
    pl.kernel,
    out_type=jax.ShapeDtypeStruct((NQ * N, 128), jnp.float32),
    mesh=_MESH,
    scratch_types=[
        pltpu.VMEM((1024,), jnp.int32),
        pltpu.VMEM((128,), jnp.int32),
        pltpu.VMEM((128,), jnp.int32),
        pltpu.VMEM((CC, 128), jnp.float32),
        pltpu.VMEM((CC, 128), jnp.float32),
        pltpu.VMEM_SHARED((ACCR, 128), jnp.float32),
        pltpu.SemaphoreType.DMA,
        pltpu.SemaphoreType.DMA,
        pltpu.SemaphoreType.DMA,
        pltpu.SemaphoreType.DMA,
    ],
)
def _sc_step1(base, to1d, zrows, out,
              tg, tA, tB, bbA, bbB, acc, bsA, bsB, ssA, ssB):
    c = lax.axis_index("c")
    s = lax.axis_index("s")
    sets = [(tA, bbA, bsA, ssA), (tB, bbB, bsB, ssB)]
    for j in range(2):
        q = 2 * c + j
        pbase = q * N
        ebase = q * EP
        for p in range(2):
            lo = p * NH
            pltpu.sync_copy(zrows, acc.at[pl.ds(s * ZR, ZR)])
            plsc.subcore_barrier()

            def group(g, _, pbase=pbase, ebase=ebase, lo=lo):
                eg = s * TILE_E + g * 1024
                pltpu.sync_copy(to1d.at[pl.ds(eg, 1024)], tg)
                t0, bb0, bs0, _s0 = sets[0]
                _stage_lidx(tg, 0, t0, lo)
                bd = [None, None]
                sd = [None, None]
                bd[0] = pltpu.async_copy(base.at[pl.ds(ebase + eg, CC)],
                                         bb0, bs0)
                for m in range(8):
                    S = m % 2
                    T = 1 - S
                    tS, bbS, _b, ssS = sets[S]
                    tT, bbT, bsT, _s2 = sets[T]
                    if m < 7:
                        if sd[T] is not None:
                            sd[T].wait()
                        _stage_lidx(tg, (m + 1) * CC, tT, lo)
                        bd[T] = pltpu.async_copy(
                            base.at[pl.ds(ebase + eg + (m + 1) * CC, CC)],
                            bbT, bsT)
                    bd[S].wait()

                    @plsc.parallel_loop(0, CC, step=1, unroll=4)
                    def row(i, bbS=bbS):
                        for u in range(8):
                            sl = pl.ds(u * 16, 16)
                            bbS[i, sl] = jnp.maximum(bbS[i, sl], 0.0)
                    sd[S] = pltpu.async_copy(bbS, acc.at[tS], ssS, add=True)
                sd[0].wait()
                sd[1].wait()
                return 0

            lax.fori_loop(0, GROUPS, group, 0)
            plsc.subcore_barrier()
            wr = 320 if p == 0 else 304
            pltpu.sync_copy(acc.at[pl.ds(s * wr, wr)],
                            out.at[pl.ds(pbase + p * NH + s * wr, wr)])
            if p == 1:
                @pl.when(s == 15)
                def _tail():
                    pltpu.sync_copy(
                        acc.at[pl.ds(16 * 304, 16)],
                        out.at[pl.ds(pbase + NH + 16 * 304, 16)])
            plsc.subcore_barrier()


def kernel(node_features, edge_features, edge_hiddens, edge_index,
           W1, W2, W3, U1, U2):
    del edge_hiddens
    from_n = edge_index[0]
    to_n = edge_index[1]
    from1d = jnp.pad(from_n, (0, EP - E))
    to1d = jnp.pad(to_n, (0, EP - E), constant_values=N)
    ef_p = jnp.pad(edge_features, ((0, EP - E), (0, 0)))
    zrows = jnp.zeros((ZR, 128), jnp.float32)

    p4 = _mm_slabs(node_features, W1.T, bm=400)
    q4 = _mm_slabs(ef_p, W2.T, bm=1024)
    base = _sc_base(p4.reshape(NQ * N, 128),
                    q4.reshape(NQ * EP, 128), from1d)

    s4 = _sc_step1(base, to1d, zrows)
    for _ in range(7):
        m4 = _mm_w3(s4.reshape(NQ, N, 128), W3.T, bm=400)
        s4 = _sc_step(base, m4.reshape(NQ * N, 128), from1d, to1d, zrows)

    return _mm_final(node_features, s4.reshape(NQ, N, 128),
                     U1.T, U2.T, bm=400)
